# baseline (speedup 1.0000x reference)
import functools

import jax
import jax.numpy as jnp
import numpy as np
from jax.experimental import pallas as pl

_LEVELS = np.array([8, 8, 8, 6, 5], dtype=np.int64)
_DIM = 768
_C = len(_LEVELS)
_CPAD = 128

_EPS = 1e-3
_levels_f = _LEVELS.astype(np.float32)
_half_l = (_levels_f - 1.0) * (1.0 + _EPS) / 2.0
_offset = np.where(_LEVELS % 2 == 0, 0.5, 0.0).astype(np.float32)
_shift = np.arctanh(_offset / _half_l).astype(np.float32)
_half_width = (_LEVELS // 2).astype(np.float32)
_basis = np.concatenate(([1], np.cumprod(_LEVELS[:-1]))).astype(np.float32)
_IDX_CONST = float(np.sum(_half_width * _basis))


def _pad_row(v, fill=0.0):
    out = np.full((_CPAD,), fill, dtype=np.float32)
    out[:_C] = v
    return out


_CVEC = np.zeros((8, _CPAD), dtype=np.float32)
_CVEC[0] = _pad_row(_shift)
_CVEC[1] = _pad_row(_half_l, fill=1.0)
_CVEC[2] = _pad_row(_offset)
_CVEC[3] = _pad_row(_basis)


def _fsq_kernel(x_ref, w_in_ref, b_in_ref, cvec_ref, w_out_ref, b_out_ref,
                idx_ref, out_ref):
    x = x_ref[...]
    z = jnp.dot(x, w_in_ref[...], preferred_element_type=jnp.float32)
    z = z + b_in_ref[0:1, :]
    bounded = (jnp.tanh(z + cvec_ref[0:1, :]) * cvec_ref[1:2, :]
               - cvec_ref[2:3, :])
    q = jnp.round(bounded)
    out_ref[...] = (jnp.dot(q, w_out_ref[...],
                            preferred_element_type=jnp.float32)
                    + b_out_ref[0:1, :])
    idx = jnp.sum(q * cvec_ref[3:4, :], axis=-1) + _IDX_CONST
    idx_ref[...] = idx.astype(jnp.int32).reshape(idx_ref.shape)


@functools.partial(jax.jit, static_argnames=("interpret",))
def kernel(x, W_in, b_in, W_out, b_out, interpret=False):
    B, T, D = x.shape
    rows = B * T
    tile = 1024
    grid = rows // tile

    xr = x.reshape(rows, D)
    w_in_p = jnp.zeros((D, _CPAD), jnp.float32).at[:, :_C].set(W_in)
    b_in_p = jnp.zeros((8, _CPAD), jnp.float32).at[0, :_C].set(b_in)
    w_out_scaled = W_out / jnp.asarray(_half_width)[:, None]
    w_out_p = jnp.zeros((_CPAD, D), jnp.float32).at[:_C, :].set(w_out_scaled)
    b_out_p = jnp.zeros((8, D), jnp.float32).at[0, :].set(b_out)
    cvec = jnp.asarray(_CVEC)

    idx3, out2 = pl.pallas_call(
        _fsq_kernel,
        grid=(grid,),
        in_specs=[
            pl.BlockSpec((tile, D), lambda i: (i, 0)),
            pl.BlockSpec((D, _CPAD), lambda i: (0, 0)),
            pl.BlockSpec((8, _CPAD), lambda i: (0, 0)),
            pl.BlockSpec((8, _CPAD), lambda i: (0, 0)),
            pl.BlockSpec((_CPAD, D), lambda i: (0, 0)),
            pl.BlockSpec((8, D), lambda i: (0, 0)),
        ],
        out_specs=[
            pl.BlockSpec((1, 1, tile), lambda i: (i, 0, 0)),
            pl.BlockSpec((tile, D), lambda i: (i, 0)),
        ],
        out_shape=[
            jax.ShapeDtypeStruct((grid, 1, tile), jnp.int32),
            jax.ShapeDtypeStruct((rows, D), jnp.float32),
        ],
        interpret=interpret,
    )(xr, w_in_p, b_in_p, cvec, w_out_p, b_out_p)

    embed_ind = idx3.reshape(B, T)
    quantize = out2.reshape(B, T, D)
    commit_loss = jnp.zeros((), dtype=jnp.float32)
    return (embed_ind, quantize, commit_loss)
